# Initial kernel scaffold; baseline (speedup 1.0000x reference)
import jax
import jax.numpy as jnp
from jax.experimental import pallas as pl


def kernel(x, edge_index, W1, b1, W2, b2, Wl, bl):
    raise NotImplementedError("write your pallas kernel here")


# speedup vs baseline: 13.7265x; 13.7265x over previous
import functools

import jax
import jax.numpy as jnp
from jax import lax
from jax.experimental import pallas as pl
from jax.experimental.pallas import tpu as pltpu
from jax.experimental.pallas import tpu_sc as plsc

N = 10000
D = 128
E = 320000

NPAD = 10240
EPAD = 327680
CHUNKS = 80
EPT = CHUNKS * 128
NROWS_PER_TILE = NPAD // 16

_mesh = plsc.VectorSubcoreMesh(core_axis_name="c", subcore_axis_name="s")


@functools.partial(
    pl.kernel,
    mesh=_mesh,
    out_type=jax.ShapeDtypeStruct((2, NPAD, 1), jnp.float32),
    compiler_params=pltpu.CompilerParams(use_tc_tiling_on_sc=False),
    scratch_types=[
        pltpu.VMEM((CHUNKS, 128), jnp.int32),
        pltpu.VMEM((128, 1), jnp.float32),
        pltpu.VMEM_SHARED((NPAD, 1), jnp.float32),
    ],
)
def _deg_kernel(dst_hbm, ones_hbm, zeros_hbm, out_hbm, dst_v, ones_v, acc_sh):
    cid = lax.axis_index("c")
    sid = lax.axis_index("s")
    wid = sid * 2 + cid

    pltpu.sync_copy(dst_hbm.at[pl.ds(wid * CHUNKS, CHUNKS)], dst_v)
    pltpu.sync_copy(ones_hbm, ones_v)
    pltpu.sync_copy(
        zeros_hbm, acc_sh.at[pl.ds(sid * NROWS_PER_TILE, NROWS_PER_TILE)]
    )
    plsc.subcore_barrier()

    def _chunk(j, _):
        pltpu.sync_copy(ones_v, acc_sh.at[dst_v.at[j]], add=True)
        return 0

    lax.fori_loop(0, CHUNKS, _chunk, 0)

    plsc.subcore_barrier()
    pltpu.sync_copy(
        acc_sh.at[pl.ds(sid * NROWS_PER_TILE, NROWS_PER_TILE)],
        out_hbm.at[cid, pl.ds(sid * NROWS_PER_TILE, NROWS_PER_TILE)],
    )


def _make_agg_kernel(width):
    @functools.partial(
        pl.kernel,
        mesh=_mesh,
        out_type=jax.ShapeDtypeStruct((2, NPAD, width), jnp.float32),
        compiler_params=pltpu.CompilerParams(use_tc_tiling_on_sc=False),
        scratch_types=[
            pltpu.VMEM((CHUNKS, 128), jnp.int32),
            pltpu.VMEM((CHUNKS, 128), jnp.int32),
            pltpu.VMEM((128, width), jnp.float32),
            pltpu.VMEM_SHARED((NPAD, width), jnp.float32),
            pltpu.SemaphoreType.DMA,
        ],
    )
    def _agg(g_hbm, src_hbm, dst_hbm, out_hbm, src_v, dst_v, rows_v, acc_sh, sem):
        cid = lax.axis_index("c")
        sid = lax.axis_index("s")
        wid = sid * 2 + cid

        pltpu.sync_copy(src_hbm.at[pl.ds(wid * CHUNKS, CHUNKS)], src_v)
        pltpu.sync_copy(dst_hbm.at[pl.ds(wid * CHUNKS, CHUNKS)], dst_v)

        zeros = jnp.zeros((16,), jnp.float32)

        def _zrow(i, _):
            for k in range(width // 16):
                rows_v[i, pl.ds(k * 16, 16)] = zeros
            return 0

        lax.fori_loop(0, 128, _zrow, 0)
        for r in range(NROWS_PER_TILE // 128):
            pltpu.sync_copy(
                rows_v, acc_sh.at[pl.ds(sid * NROWS_PER_TILE + r * 128, 128)]
            )
        plsc.subcore_barrier()

        def _chunk(j, _):
            pltpu.async_copy(g_hbm.at[src_v.at[j]], rows_v, sem).wait()
            pltpu.sync_copy(rows_v, acc_sh.at[dst_v.at[j]], add=True)
            return 0

        lax.fori_loop(0, CHUNKS, _chunk, 0)

        plsc.subcore_barrier()
        pltpu.sync_copy(
            acc_sh.at[pl.ds(sid * NROWS_PER_TILE, NROWS_PER_TILE)],
            out_hbm.at[cid, pl.ds(sid * NROWS_PER_TILE, NROWS_PER_TILE)],
        )

    return _agg


_agg32 = _make_agg_kernel(32)
_agg16 = _make_agg_kernel(16)


BLK = 1024
GRID = NPAD // BLK


def _tc1_body(x_ref, w_ref, degp_ref, g_ref):
    deg = jnp.sum(degp_ref[...], axis=0) + 1.0
    dinv = lax.rsqrt(deg)
    h = jnp.dot(x_ref[...], w_ref[...], preferred_element_type=jnp.float32)
    g_ref[...] = h * dinv[:, None]


def _tc_mid_body(aggp_ref, g_ref, degp_ref, b_ref, w_ref, out_ref):
    deg = jnp.sum(degp_ref[...], axis=0) + 1.0
    dinv = lax.rsqrt(deg)
    s = (aggp_ref[0] + aggp_ref[1] + g_ref[...]) * dinv[:, None] + b_ref[...]
    s = jnp.maximum(s, 0.0)
    h = jnp.dot(s, w_ref[...], preferred_element_type=jnp.float32)
    out_ref[...] = h * dinv[:, None]


def _tc3_body(aggp_ref, g_ref, degp_ref, b_ref, wl_ref, bl_ref, out_ref):
    deg = jnp.sum(degp_ref[...], axis=0) + 1.0
    dinv = lax.rsqrt(deg)
    s = (aggp_ref[0] + aggp_ref[1] + g_ref[...]) * dinv[:, None] + b_ref[...]
    s = jnp.maximum(s, 0.0)
    o = jnp.dot(s, wl_ref[...], preferred_element_type=jnp.float32) + bl_ref[...]
    m = jnp.max(o, axis=1, keepdims=True)
    lse = jnp.log(jnp.sum(jnp.exp(o - m), axis=1, keepdims=True)) + m
    out_ref[...] = o - lse


def _row_spec(w):
    return pl.BlockSpec((BLK, w), lambda i: (i, 0))


def _full_spec(shape):
    return pl.BlockSpec(shape, lambda i: tuple(0 for _ in shape))


def _tc1(x_pad, W1, degp):
    return pl.pallas_call(
        _tc1_body,
        grid=(GRID,),
        in_specs=[
            _row_spec(D),
            _full_spec((D, 32)),
            pl.BlockSpec((2, BLK), lambda i: (0, i)),
        ],
        out_specs=_row_spec(32),
        out_shape=jax.ShapeDtypeStruct((NPAD, 32), jnp.float32),
    )(x_pad, W1, degp)


def _tc_mid(aggp, g, degp, b, W, win, wout):
    return pl.pallas_call(
        _tc_mid_body,
        grid=(GRID,),
        in_specs=[
            pl.BlockSpec((2, BLK, win), lambda i: (0, i, 0)),
            _row_spec(win),
            pl.BlockSpec((2, BLK), lambda i: (0, i)),
            _full_spec((1, win)),
            _full_spec((win, wout)),
        ],
        out_specs=_row_spec(wout),
        out_shape=jax.ShapeDtypeStruct((NPAD, wout), jnp.float32),
    )(aggp, g, degp, b, W)


def _tc3(aggp, g, degp, b, Wl, bl):
    return pl.pallas_call(
        _tc3_body,
        grid=(GRID,),
        in_specs=[
            pl.BlockSpec((2, BLK, 16), lambda i: (0, i, 0)),
            _row_spec(16),
            pl.BlockSpec((2, BLK), lambda i: (0, i)),
            _full_spec((1, 16)),
            _full_spec((16, D)),
            _full_spec((1, D)),
        ],
        out_specs=_row_spec(D),
        out_shape=jax.ShapeDtypeStruct((NPAD, D), jnp.float32),
    )(aggp, g, degp, b, Wl, bl)


def kernel(x, edge_index, W1, b1, W2, b2, Wl, bl):
    ei = edge_index.astype(jnp.int32)
    pad = jnp.full((EPAD - E,), N, jnp.int32)
    src2d = jnp.concatenate([ei[0], pad]).reshape(EPAD // 128, 128)
    dst2d = jnp.concatenate([ei[1], pad]).reshape(EPAD // 128, 128)
    x_pad = jnp.pad(x, ((0, NPAD - N), (0, 0)))

    deg0 = jnp.zeros((NPAD,), jnp.float32).at[dst2d.reshape(-1)].add(1.0)
    degp = jnp.stack([deg0, jnp.zeros((NPAD,), jnp.float32)])

    g1 = _tc1(x_pad, W1, degp)
    agg1 = _agg32(g1, src2d, dst2d)
    g2 = _tc_mid(agg1, g1, degp, b1.reshape(1, 32), W2, 32, 16)
    agg2 = _agg16(g2, src2d, dst2d)
    out = _tc3(agg2, g2, degp, b2.reshape(1, 16), Wl, bl.reshape(1, D))
    return out[:N]


# speedup vs baseline: 25.8381x; 1.8824x over previous
import functools

import jax
import jax.numpy as jnp
from jax import lax
from jax.experimental import pallas as pl
from jax.experimental.pallas import tpu as pltpu
from jax.experimental.pallas import tpu_sc as plsc

N = 10000
D = 128
E = 320000

NPAD = 10240
EPAD = 327680
CHUNKS = 80
EPT = CHUNKS * 128
NROWS_PER_TILE = NPAD // 16

_mesh = plsc.VectorSubcoreMesh(core_axis_name="c", subcore_axis_name="s")


DEGW = 16


@functools.partial(
    pl.kernel,
    mesh=_mesh,
    out_type=jax.ShapeDtypeStruct((2, NPAD, DEGW), jnp.float32),
    compiler_params=pltpu.CompilerParams(use_tc_tiling_on_sc=False),
    scratch_types=[
        pltpu.VMEM((CHUNKS, 128), jnp.int32),
        pltpu.VMEM((128, DEGW), jnp.float32),
        pltpu.VMEM_SHARED((NPAD, DEGW), jnp.float32),
    ],
)
def _deg_kernel(dst_hbm, out_hbm, dst_v, ones_v, acc_sh):
    cid = lax.axis_index("c")
    sid = lax.axis_index("s")
    wid = sid * 2 + cid

    pltpu.sync_copy(dst_hbm.at[pl.ds(wid * CHUNKS, CHUNKS)], dst_v)

    ones = jnp.ones((16,), jnp.float32)
    zeros = jnp.zeros((16,), jnp.float32)

    def _zero(i, _):
        ones_v[i, pl.ds(0, 16)] = zeros
        return 0

    lax.fori_loop(0, 128, _zero, 0)
    for r in range(NROWS_PER_TILE // 128):
        pltpu.sync_copy(
            ones_v, acc_sh.at[pl.ds(sid * NROWS_PER_TILE + r * 128, 128)]
        )

    def _fill(i, _):
        ones_v[i, pl.ds(0, 16)] = ones
        return 0

    lax.fori_loop(0, 128, _fill, 0)
    plsc.subcore_barrier()

    def _chunk(j, _):
        pltpu.sync_copy(ones_v, acc_sh.at[dst_v.at[j]], add=True)
        return 0

    lax.fori_loop(0, CHUNKS, _chunk, 0)

    plsc.subcore_barrier()
    pltpu.sync_copy(
        acc_sh.at[pl.ds(sid * NROWS_PER_TILE, NROWS_PER_TILE)],
        out_hbm.at[cid, pl.ds(sid * NROWS_PER_TILE, NROWS_PER_TILE)],
    )


def _make_agg_kernel(width):
    @functools.partial(
        pl.kernel,
        mesh=_mesh,
        out_type=jax.ShapeDtypeStruct((2, NPAD, width), jnp.float32),
        compiler_params=pltpu.CompilerParams(use_tc_tiling_on_sc=False),
        scratch_types=[
            pltpu.VMEM((CHUNKS, 128), jnp.int32),
            pltpu.VMEM((CHUNKS, 128), jnp.int32),
            pltpu.VMEM((128, width), jnp.float32),
            pltpu.VMEM_SHARED((NPAD, width), jnp.float32),
            pltpu.SemaphoreType.DMA,
        ],
    )
    def _agg(g_hbm, src_hbm, dst_hbm, out_hbm, src_v, dst_v, rows_v, acc_sh, sem):
        cid = lax.axis_index("c")
        sid = lax.axis_index("s")
        wid = sid * 2 + cid

        pltpu.sync_copy(src_hbm.at[pl.ds(wid * CHUNKS, CHUNKS)], src_v)
        pltpu.sync_copy(dst_hbm.at[pl.ds(wid * CHUNKS, CHUNKS)], dst_v)

        zeros = jnp.zeros((16,), jnp.float32)

        def _zrow(i, _):
            for k in range(width // 16):
                rows_v[i, pl.ds(k * 16, 16)] = zeros
            return 0

        lax.fori_loop(0, 128, _zrow, 0)
        for r in range(NROWS_PER_TILE // 128):
            pltpu.sync_copy(
                rows_v, acc_sh.at[pl.ds(sid * NROWS_PER_TILE + r * 128, 128)]
            )
        plsc.subcore_barrier()

        def _chunk(j, _):
            pltpu.async_copy(g_hbm.at[src_v.at[j]], rows_v, sem).wait()
            pltpu.sync_copy(rows_v, acc_sh.at[dst_v.at[j]], add=True)
            return 0

        lax.fori_loop(0, CHUNKS, _chunk, 0)

        plsc.subcore_barrier()
        pltpu.sync_copy(
            acc_sh.at[pl.ds(sid * NROWS_PER_TILE, NROWS_PER_TILE)],
            out_hbm.at[cid, pl.ds(sid * NROWS_PER_TILE, NROWS_PER_TILE)],
        )

    return _agg


_agg32 = _make_agg_kernel(32)
_agg16 = _make_agg_kernel(16)


BLK = 1024
GRID = NPAD // BLK


def _tc1_body(x_ref, w_ref, degp_ref, g_ref):
    deg = jnp.sum(degp_ref[...], axis=0) + 1.0
    dinv = lax.rsqrt(deg)
    h = jnp.dot(x_ref[...], w_ref[...], preferred_element_type=jnp.float32)
    g_ref[...] = h * dinv[:, None]


def _tc_mid_body(aggp_ref, g_ref, degp_ref, b_ref, w_ref, out_ref):
    deg = jnp.sum(degp_ref[...], axis=0) + 1.0
    dinv = lax.rsqrt(deg)
    s = (aggp_ref[0] + aggp_ref[1] + g_ref[...]) * dinv[:, None] + b_ref[...]
    s = jnp.maximum(s, 0.0)
    h = jnp.dot(s, w_ref[...], preferred_element_type=jnp.float32)
    out_ref[...] = h * dinv[:, None]


def _tc3_body(aggp_ref, g_ref, degp_ref, b_ref, wl_ref, bl_ref, out_ref):
    deg = jnp.sum(degp_ref[...], axis=0) + 1.0
    dinv = lax.rsqrt(deg)
    s = (aggp_ref[0] + aggp_ref[1] + g_ref[...]) * dinv[:, None] + b_ref[...]
    s = jnp.maximum(s, 0.0)
    o = jnp.dot(s, wl_ref[...], preferred_element_type=jnp.float32) + bl_ref[...]
    m = jnp.max(o, axis=1, keepdims=True)
    lse = jnp.log(jnp.sum(jnp.exp(o - m), axis=1, keepdims=True)) + m
    out_ref[...] = o - lse


def _row_spec(w):
    return pl.BlockSpec((BLK, w), lambda i: (i, 0))


def _full_spec(shape):
    return pl.BlockSpec(shape, lambda i: tuple(0 for _ in shape))


def _tc1(x_pad, W1, degp):
    return pl.pallas_call(
        _tc1_body,
        grid=(GRID,),
        in_specs=[
            _row_spec(D),
            _full_spec((D, 32)),
            pl.BlockSpec((2, BLK), lambda i: (0, i)),
        ],
        out_specs=_row_spec(32),
        out_shape=jax.ShapeDtypeStruct((NPAD, 32), jnp.float32),
    )(x_pad, W1, degp)


def _tc_mid(aggp, g, degp, b, W, win, wout):
    return pl.pallas_call(
        _tc_mid_body,
        grid=(GRID,),
        in_specs=[
            pl.BlockSpec((2, BLK, win), lambda i: (0, i, 0)),
            _row_spec(win),
            pl.BlockSpec((2, BLK), lambda i: (0, i)),
            _full_spec((1, win)),
            _full_spec((win, wout)),
        ],
        out_specs=_row_spec(wout),
        out_shape=jax.ShapeDtypeStruct((NPAD, wout), jnp.float32),
    )(aggp, g, degp, b, W)


def _tc3(aggp, g, degp, b, Wl, bl):
    return pl.pallas_call(
        _tc3_body,
        grid=(GRID,),
        in_specs=[
            pl.BlockSpec((2, BLK, 16), lambda i: (0, i, 0)),
            _row_spec(16),
            pl.BlockSpec((2, BLK), lambda i: (0, i)),
            _full_spec((1, 16)),
            _full_spec((16, D)),
            _full_spec((1, D)),
        ],
        out_specs=_row_spec(D),
        out_shape=jax.ShapeDtypeStruct((NPAD, D), jnp.float32),
    )(aggp, g, degp, b, Wl, bl)


def kernel(x, edge_index, W1, b1, W2, b2, Wl, bl):
    ei = edge_index.astype(jnp.int32)
    pad = jnp.full((EPAD - E,), N, jnp.int32)
    src2d = jnp.concatenate([ei[0], pad]).reshape(EPAD // 128, 128)
    dst2d = jnp.concatenate([ei[1], pad]).reshape(EPAD // 128, 128)
    x_pad = jnp.pad(x, ((0, NPAD - N), (0, 0)))

    degp = _deg_kernel(dst2d)[:, :, 0]

    g1 = _tc1(x_pad, W1, degp)
    agg1 = _agg32(g1, src2d, dst2d)
    g2 = _tc_mid(agg1, g1, degp, b1.reshape(1, 32), W2, 32, 16)
    agg2 = _agg16(g2, src2d, dst2d)
    out = _tc3(agg2, g2, degp, b2.reshape(1, 16), Wl, bl.reshape(1, D))
    return out[:N]


# speedup vs baseline: 32.6837x; 1.2649x over previous
import functools

import jax
import jax.numpy as jnp
from jax import lax
from jax.experimental import pallas as pl
from jax.experimental.pallas import tpu as pltpu
from jax.experimental.pallas import tpu_sc as plsc

N = 10000
D = 128
E = 320000

NPAD = 10240
EPAD = 327680
CHUNKS = 80
EPT = CHUNKS * 128
NROWS_PER_TILE = NPAD // 16

_mesh = plsc.VectorSubcoreMesh(core_axis_name="c", subcore_axis_name="s")


DEGW = 16


@functools.partial(
    pl.kernel,
    mesh=_mesh,
    out_type=jax.ShapeDtypeStruct((2, NPAD, DEGW), jnp.float32),
    compiler_params=pltpu.CompilerParams(use_tc_tiling_on_sc=False),
    scratch_types=[
        pltpu.VMEM((CHUNKS, 128), jnp.int32),
        pltpu.VMEM((128, DEGW), jnp.float32),
        pltpu.VMEM_SHARED((NPAD, DEGW), jnp.float32),
    ],
)
def _deg_kernel(dst_hbm, out_hbm, dst_v, ones_v, acc_sh):
    cid = lax.axis_index("c")
    sid = lax.axis_index("s")
    wid = sid * 2 + cid

    pltpu.sync_copy(dst_hbm.at[pl.ds(wid * CHUNKS, CHUNKS)], dst_v)

    ones = jnp.ones((16,), jnp.float32)
    zeros = jnp.zeros((16,), jnp.float32)

    def _zero(i, _):
        ones_v[i, pl.ds(0, 16)] = zeros
        return 0

    lax.fori_loop(0, 128, _zero, 0)
    for r in range(NROWS_PER_TILE // 128):
        pltpu.sync_copy(
            ones_v, acc_sh.at[pl.ds(sid * NROWS_PER_TILE + r * 128, 128)]
        )

    def _fill(i, _):
        ones_v[i, pl.ds(0, 16)] = ones
        return 0

    lax.fori_loop(0, 128, _fill, 0)
    plsc.subcore_barrier()

    def _chunk(j, _):
        pltpu.sync_copy(ones_v, acc_sh.at[dst_v.at[j]], add=True)
        return 0

    lax.fori_loop(0, CHUNKS, _chunk, 0)

    plsc.subcore_barrier()
    pltpu.sync_copy(
        acc_sh.at[pl.ds(sid * NROWS_PER_TILE, NROWS_PER_TILE)],
        out_hbm.at[cid, pl.ds(sid * NROWS_PER_TILE, NROWS_PER_TILE)],
    )


def _make_agg_kernel(width):
    @functools.partial(
        pl.kernel,
        mesh=_mesh,
        out_type=jax.ShapeDtypeStruct((2, NPAD, width), jnp.float32),
        compiler_params=pltpu.CompilerParams(use_tc_tiling_on_sc=False),
        scratch_types=[
            pltpu.VMEM((CHUNKS, 128), jnp.int32),
            pltpu.VMEM((CHUNKS, 128), jnp.int32),
            pltpu.VMEM((128, width), jnp.float32),
            pltpu.VMEM((128, width), jnp.float32),
            pltpu.VMEM((128, width), jnp.float32),
            pltpu.VMEM((128, width), jnp.float32),
            pltpu.VMEM_SHARED((NPAD, width), jnp.float32),
            pltpu.SemaphoreType.DMA,
            pltpu.SemaphoreType.DMA,
            pltpu.SemaphoreType.DMA,
            pltpu.SemaphoreType.DMA,
        ],
    )
    def _agg(g_hbm, src_hbm, dst_hbm, out_hbm, src_v, dst_v, rows_a, rows_b,
             rows_c, rows_d, acc_sh, sem_a, sem_b, sem_c, sem_d):
        cid = lax.axis_index("c")
        sid = lax.axis_index("s")
        wid = sid * 2 + cid

        pltpu.sync_copy(src_hbm.at[pl.ds(wid * CHUNKS, CHUNKS)], src_v)
        pltpu.sync_copy(dst_hbm.at[pl.ds(wid * CHUNKS, CHUNKS)], dst_v)

        zeros = jnp.zeros((16,), jnp.float32)

        def _zrow(i, _):
            for k in range(width // 16):
                rows_a[i, pl.ds(k * 16, 16)] = zeros
            return 0

        lax.fori_loop(0, 128, _zrow, 0)
        for r in range(NROWS_PER_TILE // 128):
            pltpu.sync_copy(
                rows_a, acc_sh.at[pl.ds(sid * NROWS_PER_TILE + r * 128, 128)]
            )
        plsc.subcore_barrier()

        bufs = (rows_a, rows_b, rows_c, rows_d)
        sems = (sem_a, sem_b, sem_c, sem_d)
        for b in range(3):
            pltpu.async_copy(g_hbm.at[src_v.at[b]], bufs[b], sems[b])

        def _quad(i, _):
            j = i * 4
            for b in range(4):
                pltpu.make_async_copy(
                    g_hbm.at[src_v.at[j + b]], bufs[b], sems[b]
                ).wait()

                @pl.when(j + b + 3 < CHUNKS)
                def _prefetch():
                    pltpu.async_copy(
                        g_hbm.at[src_v.at[j + b + 3]],
                        bufs[(b + 3) % 4],
                        sems[(b + 3) % 4],
                    )

                pltpu.sync_copy(bufs[b], acc_sh.at[dst_v.at[j + b]], add=True)
            return 0

        lax.fori_loop(0, CHUNKS // 4, _quad, 0)

        plsc.subcore_barrier()
        pltpu.sync_copy(
            acc_sh.at[pl.ds(sid * NROWS_PER_TILE, NROWS_PER_TILE)],
            out_hbm.at[cid, pl.ds(sid * NROWS_PER_TILE, NROWS_PER_TILE)],
        )

    return _agg


_agg32 = _make_agg_kernel(32)
_agg16 = _make_agg_kernel(16)


BLK = 1024
GRID = NPAD // BLK


def _tc1_body(x_ref, w_ref, degp_ref, g_ref):
    deg = jnp.sum(degp_ref[...], axis=0) + 1.0
    dinv = lax.rsqrt(deg)
    h = jnp.dot(x_ref[...], w_ref[...], preferred_element_type=jnp.float32)
    g_ref[...] = h * dinv[:, None]


def _tc_mid_body(aggp_ref, g_ref, degp_ref, b_ref, w_ref, out_ref):
    deg = jnp.sum(degp_ref[...], axis=0) + 1.0
    dinv = lax.rsqrt(deg)
    s = (aggp_ref[0] + aggp_ref[1] + g_ref[...]) * dinv[:, None] + b_ref[...]
    s = jnp.maximum(s, 0.0)
    h = jnp.dot(s, w_ref[...], preferred_element_type=jnp.float32)
    out_ref[...] = h * dinv[:, None]


def _tc3_body(aggp_ref, g_ref, degp_ref, b_ref, wl_ref, bl_ref, out_ref):
    deg = jnp.sum(degp_ref[...], axis=0) + 1.0
    dinv = lax.rsqrt(deg)
    s = (aggp_ref[0] + aggp_ref[1] + g_ref[...]) * dinv[:, None] + b_ref[...]
    s = jnp.maximum(s, 0.0)
    o = jnp.dot(s, wl_ref[...], preferred_element_type=jnp.float32) + bl_ref[...]
    m = jnp.max(o, axis=1, keepdims=True)
    lse = jnp.log(jnp.sum(jnp.exp(o - m), axis=1, keepdims=True)) + m
    out_ref[...] = o - lse


def _row_spec(w):
    return pl.BlockSpec((BLK, w), lambda i: (i, 0))


def _full_spec(shape):
    return pl.BlockSpec(shape, lambda i: tuple(0 for _ in shape))


def _tc1(x_pad, W1, degp):
    return pl.pallas_call(
        _tc1_body,
        grid=(GRID,),
        in_specs=[
            _row_spec(D),
            _full_spec((D, 32)),
            pl.BlockSpec((2, BLK), lambda i: (0, i)),
        ],
        out_specs=_row_spec(32),
        out_shape=jax.ShapeDtypeStruct((NPAD, 32), jnp.float32),
    )(x_pad, W1, degp)


def _tc_mid(aggp, g, degp, b, W, win, wout):
    return pl.pallas_call(
        _tc_mid_body,
        grid=(GRID,),
        in_specs=[
            pl.BlockSpec((2, BLK, win), lambda i: (0, i, 0)),
            _row_spec(win),
            pl.BlockSpec((2, BLK), lambda i: (0, i)),
            _full_spec((1, win)),
            _full_spec((win, wout)),
        ],
        out_specs=_row_spec(wout),
        out_shape=jax.ShapeDtypeStruct((NPAD, wout), jnp.float32),
    )(aggp, g, degp, b, W)


def _tc3(aggp, g, degp, b, Wl, bl):
    return pl.pallas_call(
        _tc3_body,
        grid=(GRID,),
        in_specs=[
            pl.BlockSpec((2, BLK, 16), lambda i: (0, i, 0)),
            _row_spec(16),
            pl.BlockSpec((2, BLK), lambda i: (0, i)),
            _full_spec((1, 16)),
            _full_spec((16, D)),
            _full_spec((1, D)),
        ],
        out_specs=_row_spec(D),
        out_shape=jax.ShapeDtypeStruct((NPAD, D), jnp.float32),
    )(aggp, g, degp, b, Wl, bl)


def kernel(x, edge_index, W1, b1, W2, b2, Wl, bl):
    ei = edge_index.astype(jnp.int32)
    pad = jnp.full((EPAD - E,), N, jnp.int32)
    src2d = jnp.concatenate([ei[0], pad]).reshape(EPAD // 128, 128)
    dst2d = jnp.concatenate([ei[1], pad]).reshape(EPAD // 128, 128)
    x_pad = jnp.pad(x, ((0, NPAD - N), (0, 0)))

    degp = _deg_kernel(dst2d)[:, :, 0]

    g1 = _tc1(x_pad, W1, degp)
    agg1 = _agg32(g1, src2d, dst2d)
    g2 = _tc_mid(agg1, g1, degp, b1.reshape(1, 32), W2, 32, 16)
    agg2 = _agg16(g2, src2d, dst2d)
    out = _tc3(agg2, g2, degp, b2.reshape(1, 16), Wl, bl.reshape(1, D))
    return out[:N]


# speedup vs baseline: 32.9329x; 1.0076x over previous
import functools

import jax
import jax.numpy as jnp
from jax import lax
from jax.experimental import pallas as pl
from jax.experimental.pallas import tpu as pltpu
from jax.experimental.pallas import tpu_sc as plsc

N = 10000
D = 128
E = 320000

NPAD = 10240
EPAD = 327680
CHUNKS = 80
EPT = CHUNKS * 128
NROWS_PER_TILE = NPAD // 16
NBUF = 8

_mesh = plsc.VectorSubcoreMesh(core_axis_name="c", subcore_axis_name="s")


DEGW = 16


@functools.partial(
    pl.kernel,
    mesh=_mesh,
    out_type=jax.ShapeDtypeStruct((2, NPAD, DEGW), jnp.float32),
    compiler_params=pltpu.CompilerParams(use_tc_tiling_on_sc=False),
    scratch_types=[
        pltpu.VMEM((CHUNKS, 128), jnp.int32),
        pltpu.VMEM((128, DEGW), jnp.float32),
        pltpu.VMEM_SHARED((NPAD, DEGW), jnp.float32),
    ],
)
def _deg_kernel(dst_hbm, out_hbm, dst_v, ones_v, acc_sh):
    cid = lax.axis_index("c")
    sid = lax.axis_index("s")
    wid = sid * 2 + cid

    pltpu.sync_copy(dst_hbm.at[pl.ds(wid * CHUNKS, CHUNKS)], dst_v)

    ones = jnp.ones((16,), jnp.float32)
    zeros = jnp.zeros((16,), jnp.float32)

    def _zero(i, _):
        ones_v[i, pl.ds(0, 16)] = zeros
        return 0

    lax.fori_loop(0, 128, _zero, 0)
    for r in range(NROWS_PER_TILE // 128):
        pltpu.sync_copy(
            ones_v, acc_sh.at[pl.ds(sid * NROWS_PER_TILE + r * 128, 128)]
        )

    def _fill(i, _):
        ones_v[i, pl.ds(0, 16)] = ones
        return 0

    lax.fori_loop(0, 128, _fill, 0)
    plsc.subcore_barrier()

    def _chunk(j, _):
        pltpu.sync_copy(ones_v, acc_sh.at[dst_v.at[j]], add=True)
        return 0

    lax.fori_loop(0, CHUNKS, _chunk, 0)

    plsc.subcore_barrier()
    pltpu.sync_copy(
        acc_sh.at[pl.ds(sid * NROWS_PER_TILE, NROWS_PER_TILE)],
        out_hbm.at[cid, pl.ds(sid * NROWS_PER_TILE, NROWS_PER_TILE)],
    )


def _make_agg_kernel(width):
    @functools.partial(
        pl.kernel,
        mesh=_mesh,
        out_type=jax.ShapeDtypeStruct((2, NPAD, width), jnp.float32),
        compiler_params=pltpu.CompilerParams(use_tc_tiling_on_sc=False),
        scratch_types=[
            pltpu.VMEM((CHUNKS, 128), jnp.int32),
            pltpu.VMEM((CHUNKS, 128), jnp.int32),
        ]
        + [pltpu.VMEM((128, width), jnp.float32) for _ in range(NBUF)]
        + [pltpu.VMEM_SHARED((NPAD, width), jnp.float32)]
        + [pltpu.SemaphoreType.DMA for _ in range(NBUF)],
    )
    def _agg(g_hbm, src_hbm, dst_hbm, out_hbm, src_v, dst_v, *rest):
        bufs = rest[:NBUF]
        acc_sh = rest[NBUF]
        sems = rest[NBUF + 1:]
        cid = lax.axis_index("c")
        sid = lax.axis_index("s")
        wid = sid * 2 + cid

        pltpu.sync_copy(src_hbm.at[pl.ds(wid * CHUNKS, CHUNKS)], src_v)
        pltpu.sync_copy(dst_hbm.at[pl.ds(wid * CHUNKS, CHUNKS)], dst_v)

        zeros = jnp.zeros((16,), jnp.float32)

        def _zrow(i, _):
            for k in range(width // 16):
                bufs[0][i, pl.ds(k * 16, 16)] = zeros
            return 0

        lax.fori_loop(0, 128, _zrow, 0)
        for r in range(NROWS_PER_TILE // 128):
            pltpu.sync_copy(
                bufs[0], acc_sh.at[pl.ds(sid * NROWS_PER_TILE + r * 128, 128)]
            )
        plsc.subcore_barrier()

        for b in range(NBUF - 1):
            pltpu.async_copy(g_hbm.at[src_v.at[b]], bufs[b], sems[b])

        def _round(i, _):
            j = i * NBUF
            for b in range(NBUF):
                pltpu.make_async_copy(
                    g_hbm.at[src_v.at[j + b]], bufs[b], sems[b]
                ).wait()

                @pl.when(j + b + NBUF - 1 < CHUNKS)
                def _prefetch():
                    pltpu.async_copy(
                        g_hbm.at[src_v.at[j + b + NBUF - 1]],
                        bufs[(b + NBUF - 1) % NBUF],
                        sems[(b + NBUF - 1) % NBUF],
                    )

                pltpu.sync_copy(bufs[b], acc_sh.at[dst_v.at[j + b]], add=True)
            return 0

        lax.fori_loop(0, CHUNKS // NBUF, _round, 0)

        plsc.subcore_barrier()
        pltpu.sync_copy(
            acc_sh.at[pl.ds(sid * NROWS_PER_TILE, NROWS_PER_TILE)],
            out_hbm.at[cid, pl.ds(sid * NROWS_PER_TILE, NROWS_PER_TILE)],
        )

    return _agg


_agg32 = _make_agg_kernel(32)
_agg16 = _make_agg_kernel(16)


BLK = 1024
GRID = NPAD // BLK


def _tc1_body(x_ref, w_ref, degp_ref, g_ref):
    deg = jnp.sum(degp_ref[...], axis=0) + 1.0
    dinv = lax.rsqrt(deg)
    h = jnp.dot(x_ref[...], w_ref[...], preferred_element_type=jnp.float32)
    g_ref[...] = h * dinv[:, None]


def _tc_mid_body(aggp_ref, g_ref, degp_ref, b_ref, w_ref, out_ref):
    deg = jnp.sum(degp_ref[...], axis=0) + 1.0
    dinv = lax.rsqrt(deg)
    s = (aggp_ref[0] + aggp_ref[1] + g_ref[...]) * dinv[:, None] + b_ref[...]
    s = jnp.maximum(s, 0.0)
    h = jnp.dot(s, w_ref[...], preferred_element_type=jnp.float32)
    out_ref[...] = h * dinv[:, None]


def _tc3_body(aggp_ref, g_ref, degp_ref, b_ref, wl_ref, bl_ref, out_ref):
    deg = jnp.sum(degp_ref[...], axis=0) + 1.0
    dinv = lax.rsqrt(deg)
    s = (aggp_ref[0] + aggp_ref[1] + g_ref[...]) * dinv[:, None] + b_ref[...]
    s = jnp.maximum(s, 0.0)
    o = jnp.dot(s, wl_ref[...], preferred_element_type=jnp.float32) + bl_ref[...]
    m = jnp.max(o, axis=1, keepdims=True)
    lse = jnp.log(jnp.sum(jnp.exp(o - m), axis=1, keepdims=True)) + m
    out_ref[...] = o - lse


def _row_spec(w):
    return pl.BlockSpec((BLK, w), lambda i: (i, 0))


def _full_spec(shape):
    return pl.BlockSpec(shape, lambda i: tuple(0 for _ in shape))


def _tc1(x_pad, W1, degp):
    return pl.pallas_call(
        _tc1_body,
        grid=(GRID,),
        in_specs=[
            _row_spec(D),
            _full_spec((D, 32)),
            pl.BlockSpec((2, BLK), lambda i: (0, i)),
        ],
        out_specs=_row_spec(32),
        out_shape=jax.ShapeDtypeStruct((NPAD, 32), jnp.float32),
    )(x_pad, W1, degp)


def _tc_mid(aggp, g, degp, b, W, win, wout):
    return pl.pallas_call(
        _tc_mid_body,
        grid=(GRID,),
        in_specs=[
            pl.BlockSpec((2, BLK, win), lambda i: (0, i, 0)),
            _row_spec(win),
            pl.BlockSpec((2, BLK), lambda i: (0, i)),
            _full_spec((1, win)),
            _full_spec((win, wout)),
        ],
        out_specs=_row_spec(wout),
        out_shape=jax.ShapeDtypeStruct((NPAD, wout), jnp.float32),
    )(aggp, g, degp, b, W)


def _tc3(aggp, g, degp, b, Wl, bl):
    return pl.pallas_call(
        _tc3_body,
        grid=(GRID,),
        in_specs=[
            pl.BlockSpec((2, BLK, 16), lambda i: (0, i, 0)),
            _row_spec(16),
            pl.BlockSpec((2, BLK), lambda i: (0, i)),
            _full_spec((1, 16)),
            _full_spec((16, D)),
            _full_spec((1, D)),
        ],
        out_specs=_row_spec(D),
        out_shape=jax.ShapeDtypeStruct((NPAD, D), jnp.float32),
    )(aggp, g, degp, b, Wl, bl)


def kernel(x, edge_index, W1, b1, W2, b2, Wl, bl):
    ei = edge_index.astype(jnp.int32)
    pad = jnp.full((EPAD - E,), N, jnp.int32)
    src2d = jnp.concatenate([ei[0], pad]).reshape(EPAD // 128, 128)
    dst2d = jnp.concatenate([ei[1], pad]).reshape(EPAD // 128, 128)
    x_pad = jnp.pad(x, ((0, NPAD - N), (0, 0)))

    degp = _deg_kernel(dst2d)[:, :, 0]

    g1 = _tc1(x_pad, W1, degp)
    agg1 = _agg32(g1, src2d, dst2d)
    g2 = _tc_mid(agg1, g1, degp, b1.reshape(1, 32), W2, 32, 16)
    agg2 = _agg16(g2, src2d, dst2d)
    out = _tc3(agg2, g2, degp, b2.reshape(1, 16), Wl, bl.reshape(1, D))
    return out[:N]
